# baseline (speedup 1.0000x reference)
import jax
import jax.numpy as jnp
from jax import lax
from jax.experimental import pallas as pl
from jax.experimental.pallas import tpu as pltpu


def _rev8_within(data):
    h, _ = data.shape
    phase = lax.broadcasted_iota(jnp.int32, data.shape, 0)
    for s in (4, 2, 1):
        up = pltpu.roll(data, h - s, 0)
        dn = pltpu.roll(data, s, 0)
        data = jnp.where((phase & s) == 0, up, dn)
    return data


def _flip_body(mask_ref, x_ref, o_ref):
    c = pl.program_id(1)
    flag = mask_ref[c]

    @pl.when(flag == 0)
    def _copy():
        o_ref[0, 0] = x_ref[0, 0]

    @pl.when(flag != 0)
    def _flip():
        data = x_ref[0, 0]
        r8 = _rev8_within(data)
        nt = data.shape[0] // 8
        for j in range(nt):
            src = 8 * (nt - 1 - j)
            o_ref[0, 0, pl.ds(8 * j, 8)] = r8[src:src + 8]


def kernel(x, params, indices):
    B, C, H, W = x.shape
    mask = jnp.zeros((C,), jnp.int32).at[indices].set(1)
    mask = mask * params[0].astype(jnp.int32)
    grid_spec = pltpu.PrefetchScalarGridSpec(
        num_scalar_prefetch=1,
        grid=(B, C),
        in_specs=[pl.BlockSpec((1, 1, H, W), lambda b, c, mask_ref: (b, c, 0, 0))],
        out_specs=pl.BlockSpec((1, 1, H, W), lambda b, c, mask_ref: (b, c, 0, 0)),
    )
    return pl.pallas_call(
        _flip_body,
        grid_spec=grid_spec,
        out_shape=jax.ShapeDtypeStruct(x.shape, x.dtype),
        compiler_params=pltpu.CompilerParams(
            dimension_semantics=("parallel", "parallel"),
        ),
    )(mask, x)


# speedup vs baseline: 3.4536x; 3.4536x over previous
import jax
import jax.numpy as jnp
from jax import lax
from jax.experimental import pallas as pl
from jax.experimental.pallas import tpu as pltpu

_CB = 16


def _rev8_within(data, axis):
    h = data.shape[axis]
    phase = lax.broadcasted_iota(jnp.int32, data.shape, axis)
    for s in (4, 2, 1):
        up = pltpu.roll(data, h - s, axis)
        dn = pltpu.roll(data, s, axis)
        data = jnp.where((phase & s) == 0, up, dn)
    return data


def _flip_block(data):
    r8 = _rev8_within(data, 1)
    nt = data.shape[1] // 8
    return jnp.concatenate(
        [r8[:, 8 * (nt - 1 - j):8 * (nt - 1 - j) + 8] for j in range(nt)], axis=1
    )


def _flip_body(mask_ref, x_ref, maskv_ref, o_ref):
    c = pl.program_id(1)
    base = c * _CB
    count = mask_ref[base]
    for ch in range(1, _CB):
        count += mask_ref[base + ch]

    @pl.when(count == 0)
    def _copy():
        o_ref[0] = x_ref[0]

    @pl.when(count == _CB)
    def _flip_all():
        data = x_ref[0]
        r8 = _rev8_within(data, 1)
        nt = data.shape[1] // 8
        for j in range(nt):
            src = 8 * (nt - 1 - j)
            o_ref[0, :, pl.ds(8 * j, 8)] = r8[:, src:src + 8]

    @pl.when(jnp.logical_and(count > 0, count < _CB))
    def _mixed():
        data = x_ref[0]
        flipped = _flip_block(data)
        mv = maskv_ref[...][:, :, None]
        o_ref[0] = jnp.where(mv != 0, flipped, data)


def kernel(x, params, indices):
    B, C, H, W = x.shape
    mask = jnp.zeros((C,), jnp.int32).at[indices].set(1)
    mask = mask * params[0].astype(jnp.int32)
    maskv = mask.reshape(C, 1)
    grid_spec = pltpu.PrefetchScalarGridSpec(
        num_scalar_prefetch=1,
        grid=(B, C // _CB),
        in_specs=[
            pl.BlockSpec((1, _CB, H, W), lambda b, c, mask_ref: (b, c, 0, 0)),
            pl.BlockSpec((_CB, 1), lambda b, c, mask_ref: (c, 0)),
        ],
        out_specs=pl.BlockSpec((1, _CB, H, W), lambda b, c, mask_ref: (b, c, 0, 0)),
    )
    return pl.pallas_call(
        _flip_body,
        grid_spec=grid_spec,
        out_shape=jax.ShapeDtypeStruct(x.shape, x.dtype),
        compiler_params=pltpu.CompilerParams(
            dimension_semantics=("parallel", "parallel"),
        ),
    )(mask, x, maskv)


# speedup vs baseline: 3.6332x; 1.0520x over previous
import jax
import jax.numpy as jnp
from jax import lax
from jax.experimental import pallas as pl
from jax.experimental.pallas import tpu as pltpu

_CB = 24


def _rev8_within(data, axis):
    h = data.shape[axis]
    phase = lax.broadcasted_iota(jnp.int32, data.shape, axis)
    for s in (4, 2, 1):
        up = pltpu.roll(data, h - s, axis)
        dn = pltpu.roll(data, s, axis)
        data = jnp.where((phase & s) == 0, up, dn)
    return data


def _flip_block(data):
    r8 = _rev8_within(data, 1)
    nt = data.shape[1] // 8
    return jnp.concatenate(
        [r8[:, 8 * (nt - 1 - j):8 * (nt - 1 - j) + 8] for j in range(nt)], axis=1
    )


def _flip_body(mask_ref, x_ref, maskv_ref, o_ref):
    c = pl.program_id(1)
    base = c * _CB
    count = mask_ref[base]
    for ch in range(1, _CB):
        count += mask_ref[base + ch]

    @pl.when(count == 0)
    def _copy():
        o_ref[0] = x_ref[0]

    @pl.when(count == _CB)
    def _flip_all():
        data = x_ref[0]
        r8 = _rev8_within(data, 1)
        nt = data.shape[1] // 8
        for j in range(nt):
            src = 8 * (nt - 1 - j)
            o_ref[0, :, pl.ds(8 * j, 8)] = r8[:, src:src + 8]

    @pl.when(jnp.logical_and(count > 0, count < _CB))
    def _mixed():
        data = x_ref[0]
        flipped = _flip_block(data)
        mv = maskv_ref[...][:, :, None]
        o_ref[0] = jnp.where(mv != 0, flipped, data)


def kernel(x, params, indices):
    B, C, H, W = x.shape
    mask = jnp.zeros((C,), jnp.int32).at[indices].set(1)
    mask = mask * params[0].astype(jnp.int32)
    maskv = mask.reshape(C, 1)
    grid_spec = pltpu.PrefetchScalarGridSpec(
        num_scalar_prefetch=1,
        grid=(B, C // _CB),
        in_specs=[
            pl.BlockSpec((1, _CB, H, W), lambda b, c, mask_ref: (b, c, 0, 0)),
            pl.BlockSpec((_CB, 1), lambda b, c, mask_ref: (c, 0)),
        ],
        out_specs=pl.BlockSpec((1, _CB, H, W), lambda b, c, mask_ref: (b, c, 0, 0)),
    )
    return pl.pallas_call(
        _flip_body,
        grid_spec=grid_spec,
        out_shape=jax.ShapeDtypeStruct(x.shape, x.dtype),
        compiler_params=pltpu.CompilerParams(
            dimension_semantics=("parallel", "parallel"),
        ),
    )(mask, x, maskv)
